# baseline (speedup 1.0000x reference)
import functools

import jax
import jax.numpy as jnp
from jax import lax
from jax.experimental import pallas as pl
from jax.experimental.pallas import tpu as pltpu
from jax.experimental.pallas import tpu_sc as plsc

NC = 2
NS = 16
NW = NC * NS
NPAD = 10240

F32 = jnp.float32


def _sc_mesh():
  return plsc.VectorSubcoreMesh(core_axis_name="c", subcore_axis_name="s")


def _make_deg(E):
  per = E // NW
  CD = 40
  nch = per // CD

  @functools.partial(
      pl.kernel,
      mesh=_sc_mesh(),
      out_type=jax.ShapeDtypeStruct((NC * NPAD, 128), F32),
      scratch_types=[
          pltpu.VMEM((CD,), jnp.int32),
          pltpu.VMEM((CD,), jnp.int32),
          pltpu.VMEM((CD, 128), F32),
          pltpu.VMEM_SHARED((NPAD, 128), F32),
      ],
  )
  def k(src_h, dst_h, zeros_h, out_h, sidx, didx, ones_v, acc_sh):
    c = lax.axis_index("c")
    s = lax.axis_index("s")
    wid = s * NC + c

    def fill_ones(i, _):
      for j in range(8):
        ones_v[i, pl.ds(j * 16, 16)] = jnp.full((16,), 1.0, F32)
      return 0
    lax.fori_loop(0, CD, fill_ones, 0)

    pltpu.sync_copy(zeros_h.at[pl.ds(s * 640, 640)],
                    acc_sh.at[pl.ds(s * 640, 640)])
    plsc.subcore_barrier()

    def body(kk, _):
      e0 = wid * per + kk * CD
      pltpu.sync_copy(src_h.at[pl.ds(e0, CD)], sidx)
      pltpu.sync_copy(dst_h.at[pl.ds(e0, CD)], didx)
      pltpu.sync_copy(ones_v, acc_sh.at[sidx], add=True)
      pltpu.sync_copy(ones_v, acc_sh.at[didx], add=True)
      return 0
    lax.fori_loop(0, nch, body, 0)
    plsc.subcore_barrier()

    r0 = s * 640
    pltpu.sync_copy(acc_sh.at[pl.ds(r0, 640)],
                    out_h.at[pl.ds(c * NPAD + r0, 640)])

  return k


def _make_gather(E):
  per = E // NW
  CG = 40
  nch = per // CG

  @functools.partial(
      pl.kernel,
      mesh=_sc_mesh(),
      out_type=[
          jax.ShapeDtypeStruct((E, 512), F32),
          jax.ShapeDtypeStruct((NW * 2, 512), F32),
      ],
      scratch_types=[
          pltpu.VMEM((CG,), jnp.int32),
          pltpu.VMEM((CG,), jnp.int32),
          pltpu.VMEM((CG, 512), F32),
          pltpu.VMEM((CG, 512), F32),
          pltpu.VMEM((2, 512), F32),
          pltpu.SemaphoreType.DMA,
          pltpu.SemaphoreType.DMA,
      ],
  )
  def k(srcT, dstT, src_h, dst_h, h1_h, st_h,
        sidx, didx, abuf, bbuf, acc, sem1, sem2):
    c = lax.axis_index("c")
    s = lax.axis_index("s")
    wid = s * NC + c

    for j in range(32):
      acc[0, pl.ds(j * 16, 16)] = jnp.zeros((16,), F32)
      acc[1, pl.ds(j * 16, 16)] = jnp.zeros((16,), F32)

    def body(kk, _):
      e0 = wid * per + kk * CG
      pltpu.sync_copy(src_h.at[pl.ds(e0, CG)], sidx)
      pltpu.sync_copy(dst_h.at[pl.ds(e0, CG)], didx)
      ca = pltpu.async_copy(srcT.at[sidx], abuf, sem1)
      cb = pltpu.async_copy(dstT.at[didx], bbuf, sem2)
      ca.wait()
      cb.wait()

      def row(i, _):
        for j in range(32):
          sl = pl.ds(j * 16, 16)
          h = abuf[i, sl] + bbuf[i, sl]
          abuf[i, sl] = h
          acc[0, sl] = acc[0, sl] + h
          acc[1, sl] = acc[1, sl] + h * h
        return 0
      lax.fori_loop(0, CG, row, 0)

      pltpu.sync_copy(abuf, h1_h.at[pl.ds(e0, CG)])
      return 0
    lax.fori_loop(0, nch, body, 0)

    pltpu.sync_copy(acc, st_h.at[pl.ds(wid * 2, 2)])

  return k


def _make_scatter(E):
  per = E // NS
  C2 = 40
  nch = per // C2

  @functools.partial(
      pl.kernel,
      mesh=_sc_mesh(),
      out_type=jax.ShapeDtypeStruct((NC * NPAD, 128), F32),
      scratch_types=[
          pltpu.VMEM((C2,), jnp.int32),
          pltpu.VMEM((C2,), jnp.int32),
          pltpu.VMEM((C2, 256), F32),
          pltpu.VMEM((C2, 128), F32),
          pltpu.VMEM((C2, 128), F32),
          pltpu.VMEM((1, 256), F32),
          pltpu.VMEM((1, 256), F32),
          pltpu.VMEM_SHARED((NPAD, 128), F32),
      ],
  )
  def k(z2_h, src_h, dst_h, sc2_h, sh2_h, zeros_h, acc_h,
        sidx, didx, zbuf, vfi, vfo, sclv, shfv, acc_sh):
    c = lax.axis_index("c")
    s = lax.axis_index("s")

    pltpu.sync_copy(sc2_h.at[pl.ds(c, 1)], sclv)
    pltpu.sync_copy(sh2_h.at[pl.ds(c, 1)], shfv)

    pltpu.sync_copy(zeros_h.at[pl.ds(s * 640, 640)],
                    acc_sh.at[pl.ds(s * 640, 640)])
    plsc.subcore_barrier()

    def body(kk, _):
      e0 = s * per + kk * C2
      pltpu.sync_copy(src_h.at[pl.ds(e0, C2)], sidx)
      pltpu.sync_copy(dst_h.at[pl.ds(e0, C2)], didx)
      pltpu.sync_copy(z2_h.at[pl.ds(c * E + e0, C2)], zbuf)

      def row(i, _):
        for j in range(16):
          sl = pl.ds(j * 16, 16)
          v = zbuf[i, sl]
          v = jnp.maximum(v * sclv[0, sl] + shfv[0, sl], 0.0)
          if j < 8:
            vfi[i, pl.ds(j * 16, 16)] = v
          else:
            vfo[i, pl.ds((j - 8) * 16, 16)] = v
        return 0
      lax.fori_loop(0, C2, row, 0)

      pltpu.sync_copy(vfi, acc_sh.at[didx], add=True)
      pltpu.sync_copy(vfo, acc_sh.at[sidx], add=True)
      return 0
    lax.fori_loop(0, nch, body, 0)

    plsc.subcore_barrier()
    r0 = s * 640
    pltpu.sync_copy(acc_sh.at[pl.ds(r0, 640)],
                    acc_h.at[pl.ds(c * NPAD + r0, 640)])

  return k


def _make_tab(N, affine):
  BN = 1000
  G = N // BN

  def body(t_ref, sc_ref, sh_ref, wsrc_ref, wdst_ref, bsrc_ref,
           x_ref, srcT_ref, dstT_ref):
    t = t_ref[...]
    if affine:
      t = jnp.maximum(t * sc_ref[...] + sh_ref[...], 0.0)
    x_ref[...] = t
    srcT_ref[...] = jnp.dot(t, wsrc_ref[...],
                            preferred_element_type=F32) + bsrc_ref[...]
    dstT_ref[...] = jnp.dot(t, wdst_ref[...], preferred_element_type=F32)

  return pl.pallas_call(
      body,
      grid=(G,),
      in_specs=[
          pl.BlockSpec((BN, 256), lambda g: (g, 0)),
          pl.BlockSpec((1, 256), lambda g: (0, 0)),
          pl.BlockSpec((1, 256), lambda g: (0, 0)),
          pl.BlockSpec((256, 512), lambda g: (0, 0)),
          pl.BlockSpec((256, 512), lambda g: (0, 0)),
          pl.BlockSpec((1, 512), lambda g: (0, 0)),
      ],
      out_specs=[
          pl.BlockSpec((BN, 256), lambda g: (g, 0)),
          pl.BlockSpec((BN, 512), lambda g: (g, 0)),
          pl.BlockSpec((BN, 512), lambda g: (g, 0)),
      ],
      out_shape=[
          jax.ShapeDtypeStruct((N, 256), F32),
          jax.ShapeDtypeStruct((N, 512), F32),
          jax.ShapeDtypeStruct((N, 512), F32),
      ],
  )


def _make_mlp2(E):
  BE = 4000
  G = E // BE

  def body(h_ref, sc_ref, sh_ref, wfi_ref, wfo_ref, b2_ref, z2_ref, zs_ref):
    g = pl.program_id(0)
    r = jnp.maximum(h_ref[...] * sc_ref[...] + sh_ref[...], 0.0)
    zfi = jnp.dot(r[:, :256], wfi_ref[...],
                  preferred_element_type=F32) + b2_ref[0, :256]
    zfo = jnp.dot(r[:, 256:], wfo_ref[...],
                  preferred_element_type=F32) + b2_ref[0, 256:]
    z2_ref[0] = jnp.concatenate([zfi[:, :128], zfo[:, :128]], axis=1)
    z2_ref[1] = jnp.concatenate([zfi[:, 128:], zfo[:, 128:]], axis=1)
    part = jnp.stack([
        jnp.concatenate([zfi.sum(0), zfo.sum(0)]),
        jnp.concatenate([(zfi * zfi).sum(0), (zfo * zfo).sum(0)]),
    ])

    @pl.when(g == 0)
    def _():
      zs_ref[...] = part

    @pl.when(g > 0)
    def _():
      zs_ref[...] = zs_ref[...] + part

  return pl.pallas_call(
      body,
      grid=(G,),
      in_specs=[
          pl.BlockSpec((BE, 512), lambda g: (g, 0)),
          pl.BlockSpec((1, 512), lambda g: (0, 0)),
          pl.BlockSpec((1, 512), lambda g: (0, 0)),
          pl.BlockSpec((256, 256), lambda g: (0, 0)),
          pl.BlockSpec((256, 256), lambda g: (0, 0)),
          pl.BlockSpec((1, 512), lambda g: (0, 0)),
      ],
      out_specs=[
          pl.BlockSpec((2, BE, 256), lambda g: (0, g, 0)),
          pl.BlockSpec((2, 512), lambda g: (0, 0)),
      ],
      out_shape=[
          jax.ShapeDtypeStruct((2, E, 256), F32),
          jax.ShapeDtypeStruct((2, 512), F32),
      ],
  )


def _make_node(N):
  BN = 1000
  G = N // BN

  def body(x_ref, a2_ref, dv_ref, w_ref, b_ref, t_ref, ts_ref):
    g = pl.program_id(0)
    sums = jnp.concatenate([a2_ref[0], a2_ref[1]], axis=1)
    dv = dv_ref[0, :, 0:1] + dv_ref[1, :, 0:1]
    inv = jnp.where(dv > 0, 1.0 / jnp.maximum(dv, 1.0), 0.0)
    y = x_ref[...] + sums * inv
    t = jnp.dot(y, w_ref[...], preferred_element_type=F32) + b_ref[...]
    t_ref[...] = t
    part = jnp.stack([t.sum(0), (t * t).sum(0)])

    @pl.when(g == 0)
    def _():
      ts_ref[...] = part

    @pl.when(g > 0)
    def _():
      ts_ref[...] = ts_ref[...] + part

  return pl.pallas_call(
      body,
      grid=(G,),
      in_specs=[
          pl.BlockSpec((BN, 256), lambda g: (g, 0)),
          pl.BlockSpec((2, BN, 128), lambda g: (0, g, 0)),
          pl.BlockSpec((2, BN, 128), lambda g: (0, g, 0)),
          pl.BlockSpec((256, 256), lambda g: (0, 0)),
          pl.BlockSpec((1, 256), lambda g: (0, 0)),
      ],
      out_specs=[
          pl.BlockSpec((BN, 256), lambda g: (g, 0)),
          pl.BlockSpec((2, 256), lambda g: (0, 0)),
      ],
      out_shape=[
          jax.ShapeDtypeStruct((N, 256), F32),
          jax.ShapeDtypeStruct((2, 256), F32),
      ],
  )


def _make_affine(N):
  BN = 1000
  G = N // BN

  def body(t_ref, sc_ref, sh_ref, o_ref):
    o_ref[...] = jnp.maximum(t_ref[...] * sc_ref[...] + sh_ref[...], 0.0)

  return pl.pallas_call(
      body,
      grid=(G,),
      in_specs=[
          pl.BlockSpec((BN, 256), lambda g: (g, 0)),
          pl.BlockSpec((1, 256), lambda g: (0, 0)),
          pl.BlockSpec((1, 256), lambda g: (0, 0)),
      ],
      out_specs=pl.BlockSpec((BN, 256), lambda g: (g, 0)),
      out_shape=jax.ShapeDtypeStruct((N, 256), F32),
  )


def kernel(x, edge_index,
           fi_W1, fi_b1, fi_g1, fi_be1, fi_W2, fi_b2, fi_g2, fi_be2,
           fo_W1, fo_b1, fo_g1, fo_be1, fo_W2, fo_b2, fo_g2, fo_be2,
           fp_W, fp_b, fp_g, fp_be):
  N, D = x.shape
  E = edge_index.shape[1]
  src = edge_index[0]
  dst = edge_index[1]

  Wsrc = jnp.concatenate([fi_W1[:D], fo_W1[:D]], axis=1)
  Wdst = jnp.concatenate([fi_W1[D:], fo_W1[D:]], axis=1)
  bsrc = jnp.concatenate([fi_b1, fo_b1]).reshape(1, 512)
  g1 = jnp.concatenate([fi_g1, fo_g1])
  be1 = jnp.concatenate([fi_be1, fo_be1])
  b2c = jnp.concatenate([fi_b2, fo_b2]).reshape(1, 512)
  g2 = jnp.concatenate([fi_g2, fo_g2])
  be2 = jnp.concatenate([fi_be2, fo_be2])

  k_deg = _make_deg(E)
  k_gather = _make_gather(E)
  k_scatter = _make_scatter(E)
  k_tab0 = _make_tab(N, affine=False)
  k_tab1 = _make_tab(N, affine=True)
  k_mlp2 = _make_mlp2(E)
  k_node = _make_node(N)
  k_aff = _make_affine(N)

  zeros128 = jnp.zeros((NPAD, 128), F32)
  dv2 = k_deg(src, dst, zeros128).reshape(NC, NPAD, 128)

  ones_r = jnp.ones((1, 256), F32)
  zeros_r = jnp.zeros((1, 256), F32)

  t = x
  scp, shp = ones_r, zeros_r
  for step in range(2):
    if step == 0:
      xc, SRC, DST = k_tab0(t, ones_r, zeros_r, Wsrc, Wdst, bsrc)
    else:
      xc, SRC, DST = k_tab1(t, scp, shp, Wsrc, Wdst, bsrc)

    h1, hst = k_gather(SRC, DST, src, dst)
    hs = hst.reshape(NW, 2, 512).sum(0)
    m1 = hs[0] / E
    v1 = hs[1] / E - m1 * m1
    sc1 = g1 * lax.rsqrt(v1 + 1e-5)
    sh1 = be1 - m1 * sc1

    z2, zs = k_mlp2(h1, sc1.reshape(1, 512), sh1.reshape(1, 512),
                    fi_W2, fo_W2, b2c)
    m2 = zs[0] / E
    v2 = zs[1] / E - m2 * m2
    sc2 = g2 * lax.rsqrt(v2 + 1e-5)
    sh2 = be2 - m2 * sc2
    sc2r = jnp.stack([jnp.concatenate([sc2[0:128], sc2[256:384]]),
                      jnp.concatenate([sc2[128:256], sc2[384:512]])])
    sh2r = jnp.stack([jnp.concatenate([sh2[0:128], sh2[256:384]]),
                      jnp.concatenate([sh2[128:256], sh2[384:512]])])

    acc2 = k_scatter(z2.reshape(2 * E, 256), src, dst,
                     sc2r, sh2r, zeros128).reshape(NC, NPAD, 128)

    t, ts = k_node(xc, acc2, dv2, fp_W, fp_b.reshape(1, 256))
    m3 = ts[0] / N
    v3 = ts[1] / N - m3 * m3
    scp = (fp_g * lax.rsqrt(v3 + 1e-5)).reshape(1, 256)
    shp = (fp_be - ts[0] / N * scp[0]).reshape(1, 256)

  return k_aff(t, scp, shp)


# speedup vs baseline: 1.2477x; 1.2477x over previous
import functools

import jax
import jax.numpy as jnp
from jax import lax
from jax.experimental import pallas as pl
from jax.experimental.pallas import tpu as pltpu
from jax.experimental.pallas import tpu_sc as plsc

NC = 2
NS = 16
NW = NC * NS
NPAD = 10240

F32 = jnp.float32


def _sc_mesh():
  return plsc.VectorSubcoreMesh(core_axis_name="c", subcore_axis_name="s")


def _make_deg(E):
  per = E // NW
  CD = 40
  nch = per // CD

  @functools.partial(
      pl.kernel,
      mesh=_sc_mesh(),
      out_type=jax.ShapeDtypeStruct((NC * NPAD, 128), F32),
      scratch_types=[
          pltpu.VMEM((CD,), jnp.int32),
          pltpu.VMEM((CD,), jnp.int32),
          pltpu.VMEM((CD, 128), F32),
          pltpu.VMEM_SHARED((NPAD, 128), F32),
      ],
  )
  def k(src_h, dst_h, zeros_h, out_h, sidx, didx, ones_v, acc_sh):
    c = lax.axis_index("c")
    s = lax.axis_index("s")
    wid = s * NC + c

    def fill_ones(i, _):
      for j in range(8):
        ones_v[i, pl.ds(j * 16, 16)] = jnp.full((16,), 1.0, F32)
      return 0
    lax.fori_loop(0, CD, fill_ones, 0)

    pltpu.sync_copy(zeros_h.at[pl.ds(s * 640, 640)],
                    acc_sh.at[pl.ds(s * 640, 640)])
    plsc.subcore_barrier()

    def body(kk, _):
      e0 = wid * per + kk * CD
      pltpu.sync_copy(src_h.at[pl.ds(e0, CD)], sidx)
      pltpu.sync_copy(dst_h.at[pl.ds(e0, CD)], didx)
      pltpu.sync_copy(ones_v, acc_sh.at[sidx], add=True)
      pltpu.sync_copy(ones_v, acc_sh.at[didx], add=True)
      return 0
    lax.fori_loop(0, nch, body, 0)
    plsc.subcore_barrier()

    r0 = s * 640
    pltpu.sync_copy(acc_sh.at[pl.ds(r0, 640)],
                    out_h.at[pl.ds(c * NPAD + r0, 640)])

  return k


def _make_gather(E):
  per = E // NW
  CG = 40
  nch = per // CG

  @functools.partial(
      pl.kernel,
      mesh=_sc_mesh(),
      out_type=[
          jax.ShapeDtypeStruct((E, 512), F32),
          jax.ShapeDtypeStruct((NW * 2, 512), F32),
      ],
      scratch_types=[
          pltpu.VMEM((per,), jnp.int32),
          pltpu.VMEM((per,), jnp.int32),
          pltpu.VMEM((CG, 512), F32),
          pltpu.VMEM((CG, 512), F32),
          pltpu.VMEM((CG, 512), F32),
          pltpu.VMEM((CG, 512), F32),
          pltpu.VMEM((2, 512), F32),
          pltpu.SemaphoreType.DMA,
          pltpu.SemaphoreType.DMA,
      ],
  )
  def k(srcT, dstT, src_h, dst_h, h1_h, st_h,
        sidx, didx, a0, b0, a1, b1, acc, sm0, sm1):
    c = lax.axis_index("c")
    s = lax.axis_index("s")
    wid = s * NC + c

    pltpu.sync_copy(src_h.at[pl.ds(wid * per, per)], sidx)
    pltpu.sync_copy(dst_h.at[pl.ds(wid * per, per)], didx)

    for j in range(32):
      acc[0, pl.ds(j * 16, 16)] = jnp.zeros((16,), F32)
      acc[1, pl.ds(j * 16, 16)] = jnp.zeros((16,), F32)

    def fire(kk, ab, bb, sm):
      sl = pl.ds(kk * CG, CG)
      pltpu.async_copy(srcT.at[sidx.at[sl]], ab, sm)
      pltpu.async_copy(dstT.at[didx.at[sl]], bb, sm)

    def drain(ab, bb, sm):
      pltpu.make_async_copy(srcT.at[sidx.at[pl.ds(0, CG)]], ab, sm).wait()
      pltpu.make_async_copy(dstT.at[didx.at[pl.ds(0, CG)]], bb, sm).wait()

    def proc(kk, ab, bb):
      def row(i, _):
        for j in range(32):
          sl = pl.ds(j * 16, 16)
          h = ab[i, sl] + bb[i, sl]
          ab[i, sl] = h
          acc[0, sl] = acc[0, sl] + h
          acc[1, sl] = acc[1, sl] + h * h
        return 0
      lax.fori_loop(0, CG, row, 0)
      pltpu.sync_copy(ab, h1_h.at[pl.ds(wid * per + kk * CG, CG)])

    fire(0, a0, b0, sm0)

    def body(m, _):
      k0 = 2 * m
      fire(k0 + 1, a1, b1, sm1)
      drain(a0, b0, sm0)
      proc(k0, a0, b0)
      fire(k0 + 2, a0, b0, sm0)
      drain(a1, b1, sm1)
      proc(k0 + 1, a1, b1)
      return 0
    lax.fori_loop(0, (nch - 3) // 2, body, 0)

    k0 = nch - 3
    fire(k0 + 1, a1, b1, sm1)
    drain(a0, b0, sm0)
    proc(k0, a0, b0)
    fire(k0 + 2, a0, b0, sm0)
    drain(a1, b1, sm1)
    proc(k0 + 1, a1, b1)
    drain(a0, b0, sm0)
    proc(k0 + 2, a0, b0)

    pltpu.sync_copy(acc, st_h.at[pl.ds(wid * 2, 2)])

  return k


def _make_scatter(E):
  per = E // NS
  C2 = 40
  nch = per // C2

  @functools.partial(
      pl.kernel,
      mesh=_sc_mesh(),
      out_type=jax.ShapeDtypeStruct((NC * NPAD, 128), F32),
      scratch_types=[
          pltpu.VMEM((C2,), jnp.int32),
          pltpu.VMEM((C2,), jnp.int32),
          pltpu.VMEM((C2,), jnp.int32),
          pltpu.VMEM((C2,), jnp.int32),
          pltpu.VMEM((C2, 256), F32),
          pltpu.VMEM((C2, 256), F32),
          pltpu.VMEM((C2, 128), F32),
          pltpu.VMEM((C2, 128), F32),
          pltpu.VMEM((1, 256), F32),
          pltpu.VMEM((1, 256), F32),
          pltpu.VMEM_SHARED((NPAD, 128), F32),
          pltpu.SemaphoreType.DMA,
          pltpu.SemaphoreType.DMA,
      ],
  )
  def k(z2_h, src_h, dst_h, sc2_h, sh2_h, zeros_h, acc_h,
        si0, di0, si1, di1, z0, z1, vfi, vfo, sclv, shfv, acc_sh,
        sm0, sm1):
    c = lax.axis_index("c")
    s = lax.axis_index("s")

    pltpu.sync_copy(sc2_h.at[pl.ds(c, 1)], sclv)
    pltpu.sync_copy(sh2_h.at[pl.ds(c, 1)], shfv)

    pltpu.sync_copy(zeros_h.at[pl.ds(s * 640, 640)],
                    acc_sh.at[pl.ds(s * 640, 640)])
    plsc.subcore_barrier()

    def fire(kk, sib, dib, zb, sm):
      e0 = s * per + kk * C2
      pltpu.async_copy(src_h.at[pl.ds(e0, C2)], sib, sm)
      pltpu.async_copy(dst_h.at[pl.ds(e0, C2)], dib, sm)
      pltpu.async_copy(z2_h.at[pl.ds(c * E + e0, C2)], zb, sm)

    def drain(sib, dib, zb, sm):
      pltpu.make_async_copy(src_h.at[pl.ds(0, C2)], sib, sm).wait()
      pltpu.make_async_copy(dst_h.at[pl.ds(0, C2)], dib, sm).wait()
      pltpu.make_async_copy(z2_h.at[pl.ds(0, C2)], zb, sm).wait()

    def proc(sib, dib, zb):
      def row(i, _):
        for j in range(16):
          sl = pl.ds(j * 16, 16)
          v = zb[i, sl]
          v = jnp.maximum(v * sclv[0, sl] + shfv[0, sl], 0.0)
          if j < 8:
            vfi[i, pl.ds(j * 16, 16)] = v
          else:
            vfo[i, pl.ds((j - 8) * 16, 16)] = v
        return 0
      lax.fori_loop(0, C2, row, 0)

      pltpu.sync_copy(vfi, acc_sh.at[dib], add=True)
      pltpu.sync_copy(vfo, acc_sh.at[sib], add=True)

    fire(0, si0, di0, z0, sm0)

    def body(m, _):
      k0 = 2 * m
      fire(k0 + 1, si1, di1, z1, sm1)
      drain(si0, di0, z0, sm0)
      proc(si0, di0, z0)
      fire(k0 + 2, si0, di0, z0, sm0)
      drain(si1, di1, z1, sm1)
      proc(si1, di1, z1)
      return 0
    lax.fori_loop(0, (nch - 2) // 2, body, 0)

    fire(nch - 1, si1, di1, z1, sm1)
    drain(si0, di0, z0, sm0)
    proc(si0, di0, z0)
    drain(si1, di1, z1, sm1)
    proc(si1, di1, z1)

    plsc.subcore_barrier()
    r0 = s * 640
    pltpu.sync_copy(acc_sh.at[pl.ds(r0, 640)],
                    acc_h.at[pl.ds(c * NPAD + r0, 640)])

  return k


def _make_tab(N, affine):
  BN = 1000
  G = N // BN

  def body(t_ref, sc_ref, sh_ref, wsrc_ref, wdst_ref, bsrc_ref,
           x_ref, srcT_ref, dstT_ref):
    t = t_ref[...]
    if affine:
      t = jnp.maximum(t * sc_ref[...] + sh_ref[...], 0.0)
    x_ref[...] = t
    srcT_ref[...] = jnp.dot(t, wsrc_ref[...],
                            preferred_element_type=F32) + bsrc_ref[...]
    dstT_ref[...] = jnp.dot(t, wdst_ref[...], preferred_element_type=F32)

  return pl.pallas_call(
      body,
      grid=(G,),
      in_specs=[
          pl.BlockSpec((BN, 256), lambda g: (g, 0)),
          pl.BlockSpec((1, 256), lambda g: (0, 0)),
          pl.BlockSpec((1, 256), lambda g: (0, 0)),
          pl.BlockSpec((256, 512), lambda g: (0, 0)),
          pl.BlockSpec((256, 512), lambda g: (0, 0)),
          pl.BlockSpec((1, 512), lambda g: (0, 0)),
      ],
      out_specs=[
          pl.BlockSpec((BN, 256), lambda g: (g, 0)),
          pl.BlockSpec((BN, 512), lambda g: (g, 0)),
          pl.BlockSpec((BN, 512), lambda g: (g, 0)),
      ],
      out_shape=[
          jax.ShapeDtypeStruct((N, 256), F32),
          jax.ShapeDtypeStruct((N, 512), F32),
          jax.ShapeDtypeStruct((N, 512), F32),
      ],
  )


def _make_mlp2(E):
  BE = 4000
  G = E // BE

  def body(h_ref, sc_ref, sh_ref, wfi_ref, wfo_ref, b2_ref, z2_ref, zs_ref):
    g = pl.program_id(0)
    r = jnp.maximum(h_ref[...] * sc_ref[...] + sh_ref[...], 0.0)
    zfi = jnp.dot(r[:, :256], wfi_ref[...],
                  preferred_element_type=F32) + b2_ref[0, :256]
    zfo = jnp.dot(r[:, 256:], wfo_ref[...],
                  preferred_element_type=F32) + b2_ref[0, 256:]
    z2_ref[0] = jnp.concatenate([zfi[:, :128], zfo[:, :128]], axis=1)
    z2_ref[1] = jnp.concatenate([zfi[:, 128:], zfo[:, 128:]], axis=1)
    part = jnp.stack([
        jnp.concatenate([zfi.sum(0), zfo.sum(0)]),
        jnp.concatenate([(zfi * zfi).sum(0), (zfo * zfo).sum(0)]),
    ])

    @pl.when(g == 0)
    def _():
      zs_ref[...] = part

    @pl.when(g > 0)
    def _():
      zs_ref[...] = zs_ref[...] + part

  return pl.pallas_call(
      body,
      grid=(G,),
      in_specs=[
          pl.BlockSpec((BE, 512), lambda g: (g, 0)),
          pl.BlockSpec((1, 512), lambda g: (0, 0)),
          pl.BlockSpec((1, 512), lambda g: (0, 0)),
          pl.BlockSpec((256, 256), lambda g: (0, 0)),
          pl.BlockSpec((256, 256), lambda g: (0, 0)),
          pl.BlockSpec((1, 512), lambda g: (0, 0)),
      ],
      out_specs=[
          pl.BlockSpec((2, BE, 256), lambda g: (0, g, 0)),
          pl.BlockSpec((2, 512), lambda g: (0, 0)),
      ],
      out_shape=[
          jax.ShapeDtypeStruct((2, E, 256), F32),
          jax.ShapeDtypeStruct((2, 512), F32),
      ],
  )


def _make_node(N):
  BN = 1000
  G = N // BN

  def body(x_ref, a2_ref, dv_ref, w_ref, b_ref, t_ref, ts_ref):
    g = pl.program_id(0)
    sums = jnp.concatenate([a2_ref[0], a2_ref[1]], axis=1)
    dv = dv_ref[0, :, 0:1] + dv_ref[1, :, 0:1]
    inv = jnp.where(dv > 0, 1.0 / jnp.maximum(dv, 1.0), 0.0)
    y = x_ref[...] + sums * inv
    t = jnp.dot(y, w_ref[...], preferred_element_type=F32) + b_ref[...]
    t_ref[...] = t
    part = jnp.stack([t.sum(0), (t * t).sum(0)])

    @pl.when(g == 0)
    def _():
      ts_ref[...] = part

    @pl.when(g > 0)
    def _():
      ts_ref[...] = ts_ref[...] + part

  return pl.pallas_call(
      body,
      grid=(G,),
      in_specs=[
          pl.BlockSpec((BN, 256), lambda g: (g, 0)),
          pl.BlockSpec((2, BN, 128), lambda g: (0, g, 0)),
          pl.BlockSpec((2, BN, 128), lambda g: (0, g, 0)),
          pl.BlockSpec((256, 256), lambda g: (0, 0)),
          pl.BlockSpec((1, 256), lambda g: (0, 0)),
      ],
      out_specs=[
          pl.BlockSpec((BN, 256), lambda g: (g, 0)),
          pl.BlockSpec((2, 256), lambda g: (0, 0)),
      ],
      out_shape=[
          jax.ShapeDtypeStruct((N, 256), F32),
          jax.ShapeDtypeStruct((2, 256), F32),
      ],
  )


def _make_affine(N):
  BN = 1000
  G = N // BN

  def body(t_ref, sc_ref, sh_ref, o_ref):
    o_ref[...] = jnp.maximum(t_ref[...] * sc_ref[...] + sh_ref[...], 0.0)

  return pl.pallas_call(
      body,
      grid=(G,),
      in_specs=[
          pl.BlockSpec((BN, 256), lambda g: (g, 0)),
          pl.BlockSpec((1, 256), lambda g: (0, 0)),
          pl.BlockSpec((1, 256), lambda g: (0, 0)),
      ],
      out_specs=pl.BlockSpec((BN, 256), lambda g: (g, 0)),
      out_shape=jax.ShapeDtypeStruct((N, 256), F32),
  )


def kernel(x, edge_index,
           fi_W1, fi_b1, fi_g1, fi_be1, fi_W2, fi_b2, fi_g2, fi_be2,
           fo_W1, fo_b1, fo_g1, fo_be1, fo_W2, fo_b2, fo_g2, fo_be2,
           fp_W, fp_b, fp_g, fp_be):
  N, D = x.shape
  E = edge_index.shape[1]
  src = edge_index[0]
  dst = edge_index[1]

  Wsrc = jnp.concatenate([fi_W1[:D], fo_W1[:D]], axis=1)
  Wdst = jnp.concatenate([fi_W1[D:], fo_W1[D:]], axis=1)
  bsrc = jnp.concatenate([fi_b1, fo_b1]).reshape(1, 512)
  g1 = jnp.concatenate([fi_g1, fo_g1])
  be1 = jnp.concatenate([fi_be1, fo_be1])
  b2c = jnp.concatenate([fi_b2, fo_b2]).reshape(1, 512)
  g2 = jnp.concatenate([fi_g2, fo_g2])
  be2 = jnp.concatenate([fi_be2, fo_be2])

  k_deg = _make_deg(E)
  k_gather = _make_gather(E)
  k_scatter = _make_scatter(E)
  k_tab0 = _make_tab(N, affine=False)
  k_tab1 = _make_tab(N, affine=True)
  k_mlp2 = _make_mlp2(E)
  k_node = _make_node(N)
  k_aff = _make_affine(N)

  zeros128 = jnp.zeros((NPAD, 128), F32)
  dv2 = k_deg(src, dst, zeros128).reshape(NC, NPAD, 128)

  ones_r = jnp.ones((1, 256), F32)
  zeros_r = jnp.zeros((1, 256), F32)

  t = x
  scp, shp = ones_r, zeros_r
  for step in range(2):
    if step == 0:
      xc, SRC, DST = k_tab0(t, ones_r, zeros_r, Wsrc, Wdst, bsrc)
    else:
      xc, SRC, DST = k_tab1(t, scp, shp, Wsrc, Wdst, bsrc)

    h1, hst = k_gather(SRC, DST, src, dst)
    hs = hst.reshape(NW, 2, 512).sum(0)
    m1 = hs[0] / E
    v1 = hs[1] / E - m1 * m1
    sc1 = g1 * lax.rsqrt(v1 + 1e-5)
    sh1 = be1 - m1 * sc1

    z2, zs = k_mlp2(h1, sc1.reshape(1, 512), sh1.reshape(1, 512),
                    fi_W2, fo_W2, b2c)
    m2 = zs[0] / E
    v2 = zs[1] / E - m2 * m2
    sc2 = g2 * lax.rsqrt(v2 + 1e-5)
    sh2 = be2 - m2 * sc2
    sc2r = jnp.stack([jnp.concatenate([sc2[0:128], sc2[256:384]]),
                      jnp.concatenate([sc2[128:256], sc2[384:512]])])
    sh2r = jnp.stack([jnp.concatenate([sh2[0:128], sh2[256:384]]),
                      jnp.concatenate([sh2[128:256], sh2[384:512]])])

    acc2 = k_scatter(z2.reshape(2 * E, 256), src, dst,
                     sc2r, sh2r, zeros128).reshape(NC, NPAD, 128)

    t, ts = k_node(xc, acc2, dv2, fp_W, fp_b.reshape(1, 256))
    m3 = ts[0] / N
    v3 = ts[1] / N - m3 * m3
    scp = (fp_g * lax.rsqrt(v3 + 1e-5)).reshape(1, 256)
    shp = (fp_be - ts[0] / N * scp[0]).reshape(1, 256)

  return k_aff(t, scp, shp)


# speedup vs baseline: 3.7063x; 2.9705x over previous
import functools

import jax
import jax.numpy as jnp
from jax import lax
from jax.experimental import pallas as pl
from jax.experimental.pallas import tpu as pltpu
from jax.experimental.pallas import tpu_sc as plsc

NC = 2
NS = 16
NW = NC * NS
NPAD = 10240

F32 = jnp.float32


def _sc_mesh():
  return plsc.VectorSubcoreMesh(core_axis_name="c", subcore_axis_name="s")


def _make_deg(E):
  per = E // NW
  CD = 40
  nch = per // CD

  @functools.partial(
      pl.kernel,
      mesh=_sc_mesh(),
      out_type=jax.ShapeDtypeStruct((NC * NPAD, 128), F32),
      scratch_types=[
          pltpu.VMEM((CD,), jnp.int32),
          pltpu.VMEM((CD,), jnp.int32),
          pltpu.VMEM((CD, 128), F32),
          pltpu.VMEM_SHARED((NPAD, 128), F32),
      ],
  )
  def k(src_h, dst_h, zeros_h, out_h, sidx, didx, ones_v, acc_sh):
    c = lax.axis_index("c")
    s = lax.axis_index("s")
    wid = s * NC + c

    def fill_ones(i, _):
      for j in range(8):
        ones_v[i, pl.ds(j * 16, 16)] = jnp.full((16,), 1.0, F32)
      return 0
    lax.fori_loop(0, CD, fill_ones, 0)

    pltpu.sync_copy(zeros_h.at[pl.ds(s * 640, 640)],
                    acc_sh.at[pl.ds(s * 640, 640)])
    plsc.subcore_barrier()

    def body(kk, _):
      e0 = wid * per + kk * CD
      pltpu.sync_copy(src_h.at[pl.ds(e0, CD)], sidx)
      pltpu.sync_copy(dst_h.at[pl.ds(e0, CD)], didx)
      pltpu.sync_copy(ones_v, acc_sh.at[sidx], add=True)
      pltpu.sync_copy(ones_v, acc_sh.at[didx], add=True)
      return 0
    lax.fori_loop(0, nch, body, 0)
    plsc.subcore_barrier()

    r0 = s * 640
    pltpu.sync_copy(acc_sh.at[pl.ds(r0, 640)],
                    out_h.at[pl.ds(c * NPAD + r0, 640)])

  return k


def _make_gather(E):
  per = E // NW
  CG = 40
  nch = per // CG

  @functools.partial(
      pl.kernel,
      mesh=_sc_mesh(),
      out_type=jax.ShapeDtypeStruct((E, 512), F32),
      scratch_types=[
          pltpu.VMEM((per,), jnp.int32),
          pltpu.VMEM((per,), jnp.int32),
          pltpu.VMEM((CG, 512), F32),
          pltpu.VMEM((CG, 512), F32),
          pltpu.VMEM((CG, 512), F32),
          pltpu.VMEM((CG, 512), F32),
          pltpu.SemaphoreType.DMA,
          pltpu.SemaphoreType.DMA,
      ],
  )
  def k(srcT, dstT, src_h, dst_h, h1_h,
        sidx, didx, a0, b0, a1, b1, sm0, sm1):
    c = lax.axis_index("c")
    s = lax.axis_index("s")
    wid = s * NC + c

    pltpu.sync_copy(src_h.at[pl.ds(wid * per, per)], sidx)
    pltpu.sync_copy(dst_h.at[pl.ds(wid * per, per)], didx)

    def fire(kk, ab, bb, sm):
      sl = pl.ds(kk * CG, CG)
      pltpu.async_copy(srcT.at[sidx.at[sl]], ab, sm)
      pltpu.async_copy(dstT.at[didx.at[sl]], bb, sm)

    def drain(ab, bb, sm):
      pltpu.make_async_copy(srcT.at[sidx.at[pl.ds(0, CG)]], ab, sm).wait()
      pltpu.make_async_copy(dstT.at[didx.at[pl.ds(0, CG)]], bb, sm).wait()

    def proc(kk, ab, bb):
      def row(i, _):
        for j in range(32):
          sl = pl.ds(j * 16, 16)
          ab[i, sl] = ab[i, sl] + bb[i, sl]
        return 0
      lax.fori_loop(0, CG, row, 0)
      pltpu.sync_copy(ab, h1_h.at[pl.ds(wid * per + kk * CG, CG)])

    fire(0, a0, b0, sm0)

    def body(m, _):
      k0 = 2 * m
      fire(k0 + 1, a1, b1, sm1)
      drain(a0, b0, sm0)
      proc(k0, a0, b0)
      fire(k0 + 2, a0, b0, sm0)
      drain(a1, b1, sm1)
      proc(k0 + 1, a1, b1)
      return 0
    lax.fori_loop(0, (nch - 3) // 2, body, 0)

    k0 = nch - 3
    fire(k0 + 1, a1, b1, sm1)
    drain(a0, b0, sm0)
    proc(k0, a0, b0)
    fire(k0 + 2, a0, b0, sm0)
    drain(a1, b1, sm1)
    proc(k0 + 1, a1, b1)
    drain(a0, b0, sm0)
    proc(k0 + 2, a0, b0)

  return k


def _make_scatter(E):
  per = E // NS
  C2 = 80
  nch = per // C2

  @functools.partial(
      pl.kernel,
      mesh=_sc_mesh(),
      out_type=jax.ShapeDtypeStruct((NC * NPAD, 128), F32),
      scratch_types=[
          pltpu.VMEM((C2,), jnp.int32),
          pltpu.VMEM((C2,), jnp.int32),
          pltpu.VMEM((C2,), jnp.int32),
          pltpu.VMEM((C2,), jnp.int32),
          pltpu.VMEM((C2, 128), F32),
          pltpu.VMEM((C2, 128), F32),
          pltpu.VMEM((C2, 128), F32),
          pltpu.VMEM((C2, 128), F32),
          pltpu.VMEM_SHARED((NPAD, 128), F32),
          pltpu.SemaphoreType.DMA,
          pltpu.SemaphoreType.DMA,
      ],
  )
  def k(y4_h, src_h, dst_h, zeros_h, acc_h,
        si0, di0, si1, di1, f0, o0, f1, o1, acc_sh, sm0, sm1):
    c = lax.axis_index("c")
    s = lax.axis_index("s")

    pltpu.sync_copy(zeros_h.at[pl.ds(s * 640, 640)],
                    acc_sh.at[pl.ds(s * 640, 640)])
    plsc.subcore_barrier()

    def fire(kk, sib, dib, fb, ob, sm):
      e0 = s * per + kk * C2
      pltpu.async_copy(src_h.at[pl.ds(e0, C2)], sib, sm)
      pltpu.async_copy(dst_h.at[pl.ds(e0, C2)], dib, sm)
      pltpu.async_copy(y4_h.at[pl.ds((2 * c) * E + e0, C2)], fb, sm)
      pltpu.async_copy(y4_h.at[pl.ds((2 * c + 1) * E + e0, C2)], ob, sm)

    def drain(sib, dib, fb, ob, sm):
      pltpu.make_async_copy(src_h.at[pl.ds(0, C2)], sib, sm).wait()
      pltpu.make_async_copy(dst_h.at[pl.ds(0, C2)], dib, sm).wait()
      pltpu.make_async_copy(y4_h.at[pl.ds(0, C2)], fb, sm).wait()
      pltpu.make_async_copy(y4_h.at[pl.ds(0, C2)], ob, sm).wait()

    def proc(sib, dib, fb, ob):
      pltpu.sync_copy(fb, acc_sh.at[dib], add=True)
      pltpu.sync_copy(ob, acc_sh.at[sib], add=True)

    fire(0, si0, di0, f0, o0, sm0)

    def body(m, _):
      k0 = 2 * m
      fire(k0 + 1, si1, di1, f1, o1, sm1)
      drain(si0, di0, f0, o0, sm0)
      proc(si0, di0, f0, o0)
      fire(k0 + 2, si0, di0, f0, o0, sm0)
      drain(si1, di1, f1, o1, sm1)
      proc(si1, di1, f1, o1)
      return 0
    lax.fori_loop(0, (nch - 3) // 2, body, 0)

    k0 = nch - 3
    fire(k0 + 1, si1, di1, f1, o1, sm1)
    drain(si0, di0, f0, o0, sm0)
    proc(si0, di0, f0, o0)
    fire(k0 + 2, si0, di0, f0, o0, sm0)
    drain(si1, di1, f1, o1, sm1)
    proc(si1, di1, f1, o1)
    drain(si0, di0, f0, o0, sm0)
    proc(si0, di0, f0, o0)

    plsc.subcore_barrier()
    r0 = s * 640
    pltpu.sync_copy(acc_sh.at[pl.ds(r0, 640)],
                    acc_h.at[pl.ds(c * NPAD + r0, 640)])

  return k


def _make_tab(N, affine):
  BN = 1000
  G = N // BN

  def body(t_ref, sc_ref, sh_ref, wsrc_ref, wdst_ref, bsrc_ref,
           x_ref, srcT_ref, dstT_ref):
    t = t_ref[...]
    if affine:
      t = jnp.maximum(t * sc_ref[...] + sh_ref[...], 0.0)
    x_ref[...] = t
    srcT_ref[...] = jnp.dot(t, wsrc_ref[...],
                            preferred_element_type=F32) + bsrc_ref[...]
    dstT_ref[...] = jnp.dot(t, wdst_ref[...], preferred_element_type=F32)

  return pl.pallas_call(
      body,
      grid=(G,),
      in_specs=[
          pl.BlockSpec((BN, 256), lambda g: (g, 0)),
          pl.BlockSpec((1, 256), lambda g: (0, 0)),
          pl.BlockSpec((1, 256), lambda g: (0, 0)),
          pl.BlockSpec((256, 512), lambda g: (0, 0)),
          pl.BlockSpec((256, 512), lambda g: (0, 0)),
          pl.BlockSpec((1, 512), lambda g: (0, 0)),
      ],
      out_specs=[
          pl.BlockSpec((BN, 256), lambda g: (g, 0)),
          pl.BlockSpec((BN, 512), lambda g: (g, 0)),
          pl.BlockSpec((BN, 512), lambda g: (g, 0)),
      ],
      out_shape=[
          jax.ShapeDtypeStruct((N, 256), F32),
          jax.ShapeDtypeStruct((N, 512), F32),
          jax.ShapeDtypeStruct((N, 512), F32),
      ],
  )


def _make_mlp2(E):
  BE = 4000
  G = E // BE

  def body(h_ref, sc_ref, sh_ref, wfi_ref, wfo_ref, b2_ref, z2_ref, zs_ref):
    g = pl.program_id(0)
    r = jnp.maximum(h_ref[...] * sc_ref[...] + sh_ref[...], 0.0)
    zfi = jnp.dot(r[:, :256], wfi_ref[...],
                  preferred_element_type=F32) + b2_ref[0, :256]
    zfo = jnp.dot(r[:, 256:], wfo_ref[...],
                  preferred_element_type=F32) + b2_ref[0, 256:]
    z2_ref[0] = jnp.concatenate([zfi[:, :128], zfo[:, :128]], axis=1)
    z2_ref[1] = jnp.concatenate([zfi[:, 128:], zfo[:, 128:]], axis=1)
    part = jnp.stack([
        jnp.concatenate([zfi.sum(0), zfo.sum(0)]),
        jnp.concatenate([(zfi * zfi).sum(0), (zfo * zfo).sum(0)]),
    ])

    @pl.when(g == 0)
    def _():
      zs_ref[...] = part

    @pl.when(g > 0)
    def _():
      zs_ref[...] = zs_ref[...] + part

  return pl.pallas_call(
      body,
      grid=(G,),
      in_specs=[
          pl.BlockSpec((BE, 512), lambda g: (g, 0)),
          pl.BlockSpec((1, 512), lambda g: (0, 0)),
          pl.BlockSpec((1, 512), lambda g: (0, 0)),
          pl.BlockSpec((256, 256), lambda g: (0, 0)),
          pl.BlockSpec((256, 256), lambda g: (0, 0)),
          pl.BlockSpec((1, 512), lambda g: (0, 0)),
      ],
      out_specs=[
          pl.BlockSpec((2, BE, 256), lambda g: (0, g, 0)),
          pl.BlockSpec((2, 512), lambda g: (0, 0)),
      ],
      out_shape=[
          jax.ShapeDtypeStruct((2, E, 256), F32),
          jax.ShapeDtypeStruct((2, 512), F32),
      ],
  )


def _make_hstats(E):
  BE = 4000
  G = E // BE

  def body(h_ref, hs_ref):
    g = pl.program_id(0)
    h = h_ref[...]
    part = jnp.stack([h.sum(0), (h * h).sum(0)])

    @pl.when(g == 0)
    def _():
      hs_ref[...] = part

    @pl.when(g > 0)
    def _():
      hs_ref[...] = hs_ref[...] + part

  return pl.pallas_call(
      body,
      grid=(G,),
      in_specs=[pl.BlockSpec((BE, 512), lambda g: (g, 0))],
      out_specs=pl.BlockSpec((2, 512), lambda g: (0, 0)),
      out_shape=jax.ShapeDtypeStruct((2, 512), F32),
  )


def _make_bnrelu(E):
  BE = 4000
  G = E // BE

  def body(z2_ref, sc_ref, sh_ref, y4_ref):
    lo = jnp.maximum(z2_ref[0] * sc_ref[0] + sh_ref[0], 0.0)
    hi = jnp.maximum(z2_ref[1] * sc_ref[1] + sh_ref[1], 0.0)
    y4_ref[0] = lo[:, :128]
    y4_ref[1] = lo[:, 128:]
    y4_ref[2] = hi[:, :128]
    y4_ref[3] = hi[:, 128:]

  return pl.pallas_call(
      body,
      grid=(G,),
      in_specs=[
          pl.BlockSpec((2, BE, 256), lambda g: (0, g, 0)),
          pl.BlockSpec((2, 256), lambda g: (0, 0)),
          pl.BlockSpec((2, 256), lambda g: (0, 0)),
      ],
      out_specs=pl.BlockSpec((4, BE, 128), lambda g: (0, g, 0)),
      out_shape=jax.ShapeDtypeStruct((4, E, 128), F32),
  )


def _make_node(N):
  BN = 1000
  G = N // BN

  def body(x_ref, a2_ref, dv_ref, w_ref, b_ref, t_ref, ts_ref):
    g = pl.program_id(0)
    sums = jnp.concatenate([a2_ref[0], a2_ref[1]], axis=1)
    dv = dv_ref[0, :, 0:1] + dv_ref[1, :, 0:1]
    inv = jnp.where(dv > 0, 1.0 / jnp.maximum(dv, 1.0), 0.0)
    y = x_ref[...] + sums * inv
    t = jnp.dot(y, w_ref[...], preferred_element_type=F32) + b_ref[...]
    t_ref[...] = t
    part = jnp.stack([t.sum(0), (t * t).sum(0)])

    @pl.when(g == 0)
    def _():
      ts_ref[...] = part

    @pl.when(g > 0)
    def _():
      ts_ref[...] = ts_ref[...] + part

  return pl.pallas_call(
      body,
      grid=(G,),
      in_specs=[
          pl.BlockSpec((BN, 256), lambda g: (g, 0)),
          pl.BlockSpec((2, BN, 128), lambda g: (0, g, 0)),
          pl.BlockSpec((2, BN, 128), lambda g: (0, g, 0)),
          pl.BlockSpec((256, 256), lambda g: (0, 0)),
          pl.BlockSpec((1, 256), lambda g: (0, 0)),
      ],
      out_specs=[
          pl.BlockSpec((BN, 256), lambda g: (g, 0)),
          pl.BlockSpec((2, 256), lambda g: (0, 0)),
      ],
      out_shape=[
          jax.ShapeDtypeStruct((N, 256), F32),
          jax.ShapeDtypeStruct((2, 256), F32),
      ],
  )


def _make_affine(N):
  BN = 1000
  G = N // BN

  def body(t_ref, sc_ref, sh_ref, o_ref):
    o_ref[...] = jnp.maximum(t_ref[...] * sc_ref[...] + sh_ref[...], 0.0)

  return pl.pallas_call(
      body,
      grid=(G,),
      in_specs=[
          pl.BlockSpec((BN, 256), lambda g: (g, 0)),
          pl.BlockSpec((1, 256), lambda g: (0, 0)),
          pl.BlockSpec((1, 256), lambda g: (0, 0)),
      ],
      out_specs=pl.BlockSpec((BN, 256), lambda g: (g, 0)),
      out_shape=jax.ShapeDtypeStruct((N, 256), F32),
  )


def kernel(x, edge_index,
           fi_W1, fi_b1, fi_g1, fi_be1, fi_W2, fi_b2, fi_g2, fi_be2,
           fo_W1, fo_b1, fo_g1, fo_be1, fo_W2, fo_b2, fo_g2, fo_be2,
           fp_W, fp_b, fp_g, fp_be):
  N, D = x.shape
  E = edge_index.shape[1]
  src = edge_index[0]
  dst = edge_index[1]

  Wsrc = jnp.concatenate([fi_W1[:D], fo_W1[:D]], axis=1)
  Wdst = jnp.concatenate([fi_W1[D:], fo_W1[D:]], axis=1)
  bsrc = jnp.concatenate([fi_b1, fo_b1]).reshape(1, 512)
  g1 = jnp.concatenate([fi_g1, fo_g1])
  be1 = jnp.concatenate([fi_be1, fo_be1])
  b2c = jnp.concatenate([fi_b2, fo_b2]).reshape(1, 512)
  g2 = jnp.concatenate([fi_g2, fo_g2])
  be2 = jnp.concatenate([fi_be2, fo_be2])

  k_deg = _make_deg(E)
  k_gather = _make_gather(E)
  k_scatter = _make_scatter(E)
  k_tab0 = _make_tab(N, affine=False)
  k_tab1 = _make_tab(N, affine=True)
  k_mlp2 = _make_mlp2(E)
  k_hstats = _make_hstats(E)
  k_bnrelu = _make_bnrelu(E)
  k_node = _make_node(N)
  k_aff = _make_affine(N)

  zeros128 = jnp.zeros((NPAD, 128), F32)
  dv2 = k_deg(src, dst, zeros128).reshape(NC, NPAD, 128)

  ones_r = jnp.ones((1, 256), F32)
  zeros_r = jnp.zeros((1, 256), F32)

  t = x
  scp, shp = ones_r, zeros_r
  for step in range(2):
    if step == 0:
      xc, SRC, DST = k_tab0(t, ones_r, zeros_r, Wsrc, Wdst, bsrc)
    else:
      xc, SRC, DST = k_tab1(t, scp, shp, Wsrc, Wdst, bsrc)

    h1 = k_gather(SRC, DST, src, dst)
    hs = k_hstats(h1)
    m1 = hs[0] / E
    v1 = hs[1] / E - m1 * m1
    sc1 = g1 * lax.rsqrt(v1 + 1e-5)
    sh1 = be1 - m1 * sc1

    z2, zs = k_mlp2(h1, sc1.reshape(1, 512), sh1.reshape(1, 512),
                    fi_W2, fo_W2, b2c)
    m2 = zs[0] / E
    v2 = zs[1] / E - m2 * m2
    sc2 = g2 * lax.rsqrt(v2 + 1e-5)
    sh2 = be2 - m2 * sc2
    sc2r = jnp.stack([jnp.concatenate([sc2[0:128], sc2[256:384]]),
                      jnp.concatenate([sc2[128:256], sc2[384:512]])])
    sh2r = jnp.stack([jnp.concatenate([sh2[0:128], sh2[256:384]]),
                      jnp.concatenate([sh2[128:256], sh2[384:512]])])

    y4 = k_bnrelu(z2, sc2r, sh2r)
    acc2 = k_scatter(y4.reshape(4 * E, 128), src, dst,
                     zeros128).reshape(NC, NPAD, 128)

    t, ts = k_node(xc, acc2, dv2, fp_W, fp_b.reshape(1, 256))
    m3 = ts[0] / N
    v3 = ts[1] / N - m3 * m3
    scp = (fp_g * lax.rsqrt(v3 + 1e-5)).reshape(1, 256)
    shp = (fp_be - ts[0] / N * scp[0]).reshape(1, 256)

  return k_aff(t, scp, shp)
